# baseline (speedup 1.0000x reference)
import functools

import jax
import jax.numpy as jnp
from jax import lax
from jax.experimental import pallas as pl
from jax.experimental.pallas import tpu as pltpu
from jax.experimental.pallas import tpu_sc as plsc

_BATCH = 4096
_HIST = 200
_EMBED = 32
_OUT = 16
_NC = 2
_NS = 16
_NW = _NC * _NS
_RPW = _BATCH // _NW
_HA = 128
_HB = _HIST - _HA


def _make_pool_kernel():
    mesh = plsc.VectorSubcoreMesh(core_axis_name="c", subcore_axis_name="s")

    @functools.partial(
        pl.kernel,
        mesh=mesh,
        compiler_params=pltpu.CompilerParams(use_tc_tiling_on_sc=False),
        out_type=jax.ShapeDtypeStruct((_BATCH * _EMBED * 4,), jnp.bfloat16),
        scratch_types=[
            pltpu.VMEM((_RPW, _HA), jnp.int32),
            pltpu.VMEM((_RPW, _HB), jnp.int32),
            pltpu.VMEM((_HA, _EMBED), jnp.bfloat16),
            pltpu.VMEM((_HA, _EMBED), jnp.bfloat16),
            pltpu.VMEM((_HB, _EMBED), jnp.bfloat16),
            pltpu.VMEM((_HB, _EMBED), jnp.bfloat16),
            pltpu.VMEM((_RPW * _EMBED * 4,), jnp.bfloat16),
            pltpu.SemaphoreType.DMA,
            pltpu.SemaphoreType.DMA,
            pltpu.SemaphoreType.DMA,
            pltpu.SemaphoreType.DMA,
        ],
    )
    def pool(x, table, out, idxa_v, idxb_v, bufA0, bufA1, bufB0,
             bufB1, out_v, semA0, semA1, semB0, semB1):
        wid = lax.axis_index("s") * _NC + lax.axis_index("c")

        rows = pl.ds(wid * _RPW, _RPW)
        pltpu.sync_copy(x.at[rows, pl.ds(0, _HA)], idxa_v)
        pltpu.sync_copy(x.at[rows, pl.ds(_HA, _HB)], idxb_v)

        def fire(r, bufA, bufB, semA, semB):
            pltpu.async_copy(table.at[idxa_v.at[r]], bufA, semA)
            pltpu.async_copy(table.at[idxb_v.at[r]], bufB, semB)

        def drain(bufA, bufB, semA, semB):
            pltpu.make_async_copy(table.at[idxa_v.at[0]], bufA, semA).wait()
            pltpu.make_async_copy(table.at[idxb_v.at[0]], bufB, semB).wait()

        def accum(r, bufA, bufB):
            z = jnp.zeros((2 * 16,), jnp.bfloat16)

            @plsc.parallel_loop(0, _HA, step=8, carry=(z,) * 4)
            def accA(j, c4):
                p = list(c4)
                for t in range(8):
                    p[t % 4] = p[t % 4] + bufA[j + t, 0:_EMBED]
                return tuple(p)

            @plsc.parallel_loop(0, _HB, step=8, carry=accA)
            def accB(j, c4):
                p = list(c4)
                for t in range(8):
                    p[t % 4] = p[t % 4] + bufB[j + t, 0:_EMBED]
                return tuple(p)

            for t in range(4):
                out_v[pl.ds((r * 4 + t) * _EMBED, _EMBED)] = accB[t]

        fire(0, bufA0, bufB0, semA0, semB0)

        def body(i, carry):
            r0 = 2 * i
            fire(r0 + 1, bufA1, bufB1, semA1, semB1)
            drain(bufA0, bufB0, semA0, semB0)
            accum(r0, bufA0, bufB0)

            @pl.when(i < _RPW // 2 - 1)
            def _():
                fire(r0 + 2, bufA0, bufB0, semA0, semB0)

            drain(bufA1, bufB1, semA1, semB1)
            accum(r0 + 1, bufA1, bufB1)
            return carry

        lax.fori_loop(0, _RPW // 2, body, 0)
        pltpu.sync_copy(out_v, out.at[pl.ds(wid * _RPW * _EMBED * 4,
                                            _RPW * _EMBED * 4)])

    return pool


_pool_kernel = _make_pool_kernel()


_VOCAB = 1000000
_NCOLS = _VOCAB // 128
_TAILROWS = _VOCAB - _NCOLS * 128
_UL = 256
_NUNITS = _NCOLS * 128 // _UL
_KTOT = 124


def _make_detile_kernel():
    mesh = plsc.VectorSubcoreMesh(core_axis_name="c", subcore_axis_name="s")

    @functools.partial(
        pl.kernel,
        mesh=mesh,
        compiler_params=pltpu.CompilerParams(use_tc_tiling_on_sc=True,
                                             needs_layout_passes=False),
        out_type=jax.ShapeDtypeStruct((_VOCAB * _EMBED // 2,), jnp.int32),
        scratch_types=[
            pltpu.VMEM((_EMBED, _UL), jnp.float32),
            pltpu.VMEM((_EMBED, _UL), jnp.float32),
            pltpu.VMEM((_UL * _EMBED // 2,), jnp.int32),
            pltpu.VMEM((_UL * _EMBED // 2,), jnp.int32),
            pltpu.VMEM((_UL * 33,), jnp.float32),
            pltpu.VMEM((_TAILROWS * _EMBED // 2,), jnp.int32),
            pltpu.SemaphoreType.DMA,
            pltpu.SemaphoreType.DMA,
            pltpu.SemaphoreType.DMA,
            pltpu.SemaphoreType.DMA,
        ],
    )
    def detile(tT, tail, out, tile0, tile1, stage0, stage1, skew_v, tailb,
               si0, si1, so0, so1):
        wid = lax.axis_index("s") * _NC + lax.axis_index("c")
        lane33 = lax.iota(jnp.int32, 16) * 33

        def colof(k):
            return jnp.minimum(k * _NW + wid, _NUNITS - 1)

        def fire_in(u, tile_v, sem):
            pltpu.async_copy(tT.at[:, pl.ds(u * _UL, _UL)], tile_v, sem)

        def wait_in(tile_v, sem):
            pltpu.make_async_copy(tT.at[:, pl.ds(0, _UL)], tile_v,
                                  sem).wait()

        def fire_out(u, stage_v, sem):
            off = pl.multiple_of(u * (_UL * _EMBED // 2), 8)
            pltpu.async_copy(stage_v,
                             out.at[pl.ds(off, _UL * _EMBED // 2)], sem)

        def wait_out(stage_v, sem):
            pltpu.make_async_copy(stage_v,
                                  out.at[pl.ds(0, _UL * _EMBED // 2)],
                                  sem).wait()

        def shuffle(tile_v, stage_v):
            @plsc.parallel_loop(0, _EMBED, unroll=4)
            def _(c):
                for g in range(_UL // 16):
                    v = tile_v[c, 16 * g:16 * (g + 1)]
                    plsc.store_scatter(skew_v, [lane33 + (33 * 16 * g) + c],
                                       v)

            @plsc.parallel_loop(0, _UL, unroll=8)
            def _(r):
                a = skew_v[pl.ds(r * 33, 16)]
                b = skew_v[pl.ds(r * 33 + 16, 16)]
                packed = plsc.pack(a, b, format=plsc.PackFormat.INTERLEAVED)
                stage_v[pl.ds(r * 16, 16)] = plsc.bitcast(packed, jnp.int32)

        fire_in(colof(0), tile0, si0)
        fire_in(colof(1), tile1, si1)

        def body(i, carry):
            k0 = 2 * i
            wait_in(tile0, si0)

            @pl.when(i > 0)
            def _():
                wait_out(stage0, so0)

            shuffle(tile0, stage0)
            fire_out(colof(k0), stage0, so0)
            fire_in(colof(k0 + 2), tile0, si0)

            wait_in(tile1, si1)

            @pl.when(i > 0)
            def _():
                wait_out(stage1, so1)

            shuffle(tile1, stage1)
            fire_out(colof(k0 + 1), stage1, so1)
            fire_in(colof(k0 + 3), tile1, si1)
            return carry

        lax.fori_loop(0, _KTOT // 2, body, 0)
        wait_in(tile0, si0)
        wait_in(tile1, si1)
        wait_out(stage0, so0)
        wait_out(stage1, so1)

        @pl.when(wid == 0)
        def _():
            pltpu.sync_copy(tail, tailb)
            pltpu.sync_copy(
                tailb, out.at[pl.ds(_NCOLS * 128 * _EMBED // 2,
                                    _TAILROWS * _EMBED // 2)])

    return detile


_detile_kernel = _make_detile_kernel()


def _mm_body(p_ref, w_ref, b_ref, o_ref):
    x = p_ref[...].astype(jnp.float32)
    pooled = ((x[:, 0:_EMBED] + x[:, _EMBED:2 * _EMBED])
              + (x[:, 2 * _EMBED:3 * _EMBED] + x[:, 3 * _EMBED:4 * _EMBED]))
    o_ref[...] = (
        jnp.dot(pooled, w_ref[...], preferred_element_type=jnp.float32)
        + b_ref[...]
    )


_ILEAVE = [v for i in range(16) for v in (i, 16 + i)]


def kernel(x, table, W, b):
    xi = x.astype(jnp.int32)
    tail_bf = (table[_NCOLS * 128:, _ILEAVE].astype(jnp.bfloat16)
               .reshape(_TAILROWS * _EMBED // 2, 2))
    tail = jax.lax.bitcast_convert_type(tail_bf, jnp.int32)
    t_i32 = _detile_kernel(table.T, tail)
    t_lin = jax.lax.bitcast_convert_type(
        t_i32, jnp.bfloat16).reshape(_VOCAB, _EMBED)
    pooled4 = _pool_kernel(xi, t_lin).reshape(_BATCH, 4 * _EMBED)
    wt = (W.T[_ILEAVE, :] / float(_HIST)).astype(jnp.float32)
    out = pl.pallas_call(
        _mm_body,
        out_shape=jax.ShapeDtypeStruct((_BATCH, _OUT), jnp.float32),
    )(pooled4, wt, b.reshape(1, _OUT))
    return out


# speedup vs baseline: 5.0584x; 5.0584x over previous
import functools

import jax
import jax.numpy as jnp
from jax import lax
from jax.experimental import pallas as pl
from jax.experimental.pallas import tpu as pltpu
from jax.experimental.pallas import tpu_sc as plsc

_BATCH = 4096
_HIST = 200
_EMBED = 32
_OUT = 16
_NC = 2
_NS = 16
_NW = _NC * _NS
_RPW = _BATCH // _NW
_HA = 128
_HB = _HIST - _HA


def _make_pool_kernel():
    mesh = plsc.VectorSubcoreMesh(core_axis_name="c", subcore_axis_name="s")

    @functools.partial(
        pl.kernel,
        mesh=mesh,
        compiler_params=pltpu.CompilerParams(use_tc_tiling_on_sc=False,
                                             needs_layout_passes=False),
        out_type=jax.ShapeDtypeStruct((_BATCH * _EMBED,), jnp.float32),
        scratch_types=[
            pltpu.VMEM((_RPW, _HA), jnp.int32),
            pltpu.VMEM((_RPW, _HB), jnp.int32),
            pltpu.VMEM((_HA, _EMBED // 2), jnp.int32),
            pltpu.VMEM((_HA, _EMBED // 2), jnp.int32),
            pltpu.VMEM((_HB, _EMBED // 2), jnp.int32),
            pltpu.VMEM((_HB, _EMBED // 2), jnp.int32),
            pltpu.VMEM((_RPW * _EMBED,), jnp.float32),
            pltpu.SemaphoreType.DMA,
            pltpu.SemaphoreType.DMA,
            pltpu.SemaphoreType.DMA,
            pltpu.SemaphoreType.DMA,
        ],
    )
    def pool(x, table, out, idxa_v, idxb_v, bufA0, bufA1, bufB0,
             bufB1, out_v, semA0, semA1, semB0, semB1):
        wid = lax.axis_index("s") * _NC + lax.axis_index("c")

        rows = pl.ds(wid * _RPW, _RPW)
        pltpu.sync_copy(x.at[rows, pl.ds(0, _HA)], idxa_v)
        pltpu.sync_copy(x.at[rows, pl.ds(_HA, _HB)], idxb_v)

        def fire(r, bufA, bufB, semA, semB):
            pltpu.async_copy(table.at[idxa_v.at[r]], bufA, semA)
            pltpu.async_copy(table.at[idxb_v.at[r]], bufB, semB)

        def drain(bufA, bufB, semA, semB):
            pltpu.make_async_copy(table.at[idxa_v.at[0]], bufA, semA).wait()
            pltpu.make_async_copy(table.at[idxb_v.at[0]], bufB, semB).wait()

        def accum(r, bufA, bufB):
            z = jnp.zeros((2 * 16,), jnp.bfloat16)

            @plsc.parallel_loop(0, _HA, step=8, carry=(z,) * 4)
            def accA(j, c4):
                p = list(c4)
                for t in range(8):
                    v = plsc.bitcast(bufA[j + t, 0:16], jnp.bfloat16)
                    p[t % 4] = p[t % 4] + v
                return tuple(p)

            @plsc.parallel_loop(0, _HB, step=8, carry=accA)
            def accB(j, c4):
                p = list(c4)
                for t in range(8):
                    v = plsc.bitcast(bufB[j + t, 0:16], jnp.bfloat16)
                    p[t % 4] = p[t % 4] + v
                return tuple(p)

            p0, p1, p2, p3 = accB
            stot = (p0 + p1) + (p2 + p3)
            a, b2 = plsc.unpack(stot, format=plsc.PackFormat.INTERLEAVED)
            out_v[pl.ds(r * _EMBED, 16)] = a
            out_v[pl.ds(r * _EMBED + 16, 16)] = b2

        fire(0, bufA0, bufB0, semA0, semB0)

        def body(i, carry):
            r0 = 2 * i
            fire(r0 + 1, bufA1, bufB1, semA1, semB1)
            drain(bufA0, bufB0, semA0, semB0)
            accum(r0, bufA0, bufB0)

            @pl.when(i < _RPW // 2 - 1)
            def _():
                fire(r0 + 2, bufA0, bufB0, semA0, semB0)

            drain(bufA1, bufB1, semA1, semB1)
            accum(r0 + 1, bufA1, bufB1)
            return carry

        lax.fori_loop(0, _RPW // 2, body, 0)
        pltpu.sync_copy(out_v, out.at[pl.ds(wid * _RPW * _EMBED,
                                            _RPW * _EMBED)])

    return pool


_pool_kernel = _make_pool_kernel()


_VOCAB = 1000000
_NCOLS = _VOCAB // 128
_TAILROWS = _VOCAB - _NCOLS * 128
_UL = 256
_NUNITS = _NCOLS * 128 // _UL
_KTOT = 124


def _make_detile_kernel():
    mesh = plsc.VectorSubcoreMesh(core_axis_name="c", subcore_axis_name="s")

    @functools.partial(
        pl.kernel,
        mesh=mesh,
        compiler_params=pltpu.CompilerParams(use_tc_tiling_on_sc=True,
                                             needs_layout_passes=False),
        out_type=jax.ShapeDtypeStruct((_VOCAB * _EMBED // 2,), jnp.int32),
        scratch_types=[
            pltpu.VMEM((_EMBED, _UL), jnp.float32),
            pltpu.VMEM((_EMBED, _UL), jnp.float32),
            pltpu.VMEM((_UL * _EMBED // 2,), jnp.int32),
            pltpu.VMEM((_UL * _EMBED // 2,), jnp.int32),
            pltpu.VMEM((_UL * 33,), jnp.float32),
            pltpu.VMEM((_TAILROWS * _EMBED // 2,), jnp.int32),
            pltpu.SemaphoreType.DMA,
            pltpu.SemaphoreType.DMA,
            pltpu.SemaphoreType.DMA,
            pltpu.SemaphoreType.DMA,
        ],
    )
    def detile(tT, tail, out, tile0, tile1, stage0, stage1, skew_v, tailb,
               si0, si1, so0, so1):
        wid = lax.axis_index("s") * _NC + lax.axis_index("c")
        lane33 = lax.iota(jnp.int32, 16) * 33

        def colof(k):
            return jnp.minimum(k * _NW + wid, _NUNITS - 1)

        def fire_in(u, tile_v, sem):
            pltpu.async_copy(tT.at[:, pl.ds(u * _UL, _UL)], tile_v, sem)

        def wait_in(tile_v, sem):
            pltpu.make_async_copy(tT.at[:, pl.ds(0, _UL)], tile_v,
                                  sem).wait()

        def fire_out(u, stage_v, sem):
            off = pl.multiple_of(u * (_UL * _EMBED // 2), 8)
            pltpu.async_copy(stage_v,
                             out.at[pl.ds(off, _UL * _EMBED // 2)], sem)

        def wait_out(stage_v, sem):
            pltpu.make_async_copy(stage_v,
                                  out.at[pl.ds(0, _UL * _EMBED // 2)],
                                  sem).wait()

        def shuffle(tile_v, stage_v):
            @plsc.parallel_loop(0, _EMBED, unroll=4)
            def _(c):
                for g in range(_UL // 16):
                    v = tile_v[c, 16 * g:16 * (g + 1)]
                    plsc.store_scatter(skew_v, [lane33 + (33 * 16 * g) + c],
                                       v)

            @plsc.parallel_loop(0, _UL, unroll=8)
            def _(r):
                a = skew_v[pl.ds(r * 33, 16)]
                b = skew_v[pl.ds(r * 33 + 16, 16)]
                packed = plsc.pack(a, b, format=plsc.PackFormat.INTERLEAVED)
                stage_v[pl.ds(r * 16, 16)] = plsc.bitcast(packed, jnp.int32)

        fire_in(colof(0), tile0, si0)
        fire_in(colof(1), tile1, si1)

        def body(i, carry):
            k0 = 2 * i
            wait_in(tile0, si0)

            @pl.when(i > 0)
            def _():
                wait_out(stage0, so0)

            shuffle(tile0, stage0)
            fire_out(colof(k0), stage0, so0)
            fire_in(colof(k0 + 2), tile0, si0)

            wait_in(tile1, si1)

            @pl.when(i > 0)
            def _():
                wait_out(stage1, so1)

            shuffle(tile1, stage1)
            fire_out(colof(k0 + 1), stage1, so1)
            fire_in(colof(k0 + 3), tile1, si1)
            return carry

        lax.fori_loop(0, _KTOT // 2, body, 0)
        wait_in(tile0, si0)
        wait_in(tile1, si1)
        wait_out(stage0, so0)
        wait_out(stage1, so1)

        @pl.when(wid == 0)
        def _():
            pltpu.sync_copy(tail, tailb)
            pltpu.sync_copy(
                tailb, out.at[pl.ds(_NCOLS * 128 * _EMBED // 2,
                                    _TAILROWS * _EMBED // 2)])

    return detile


_detile_kernel = _make_detile_kernel()


def _mm_body(p_ref, w_ref, b_ref, o_ref):
    o_ref[...] = (
        jnp.dot(p_ref[...], w_ref[...], preferred_element_type=jnp.float32)
        + b_ref[...]
    )


_ILEAVE = [v for i in range(16) for v in (i, 16 + i)]


def kernel(x, table, W, b):
    xi = x.astype(jnp.int32)
    tail_bf = (table[_NCOLS * 128:, _ILEAVE].astype(jnp.bfloat16)
               .reshape(_TAILROWS * _EMBED // 2, 2))
    tail = jax.lax.bitcast_convert_type(tail_bf, jnp.int32)
    t16 = _detile_kernel(table.T, tail).reshape(_VOCAB, _EMBED // 2)
    pooled = _pool_kernel(xi, t16).reshape(_BATCH, _EMBED)
    wt = (W.T / float(_HIST)).astype(jnp.float32)
    out = pl.pallas_call(
        _mm_body,
        out_shape=jax.ShapeDtypeStruct((_BATCH, _OUT), jnp.float32),
    )(pooled, wt, b.reshape(1, _OUT))
    return out


# speedup vs baseline: 8.8527x; 1.7501x over previous
import functools

import jax
import jax.numpy as jnp
from jax import lax
from jax.experimental import pallas as pl
from jax.experimental.pallas import tpu as pltpu
from jax.experimental.pallas import tpu_sc as plsc

_BATCH = 4096
_HIST = 200
_EMBED = 32
_OUT = 16
_NC = 2
_NS = 16
_NW = _NC * _NS
_RPW = _BATCH // _NW
_HA = 128
_HB = _HIST - _HA


def _make_pool_kernel():
    mesh = plsc.VectorSubcoreMesh(core_axis_name="c", subcore_axis_name="s")

    @functools.partial(
        pl.kernel,
        mesh=mesh,
        compiler_params=pltpu.CompilerParams(use_tc_tiling_on_sc=False,
                                             needs_layout_passes=False),
        out_type=jax.ShapeDtypeStruct((_BATCH * _EMBED,), jnp.float32),
        scratch_types=[
            pltpu.VMEM((_RPW, _HA), jnp.int32),
            pltpu.VMEM((_RPW, _HB), jnp.int32),
            pltpu.VMEM((_HA, _EMBED // 2), jnp.int32),
            pltpu.VMEM((_HA, _EMBED // 2), jnp.int32),
            pltpu.VMEM((_HB, _EMBED // 2), jnp.int32),
            pltpu.VMEM((_HB, _EMBED // 2), jnp.int32),
            pltpu.VMEM((_RPW * _EMBED,), jnp.float32),
            pltpu.SemaphoreType.DMA,
            pltpu.SemaphoreType.DMA,
            pltpu.SemaphoreType.DMA,
            pltpu.SemaphoreType.DMA,
        ],
    )
    def pool(x, table, out, idxa_v, idxb_v, bufA0, bufA1, bufB0,
             bufB1, out_v, semA0, semA1, semB0, semB1):
        wid = lax.axis_index("s") * _NC + lax.axis_index("c")

        rows = pl.ds(wid * _RPW, _RPW)
        pltpu.sync_copy(x.at[rows, pl.ds(0, _HA)], idxa_v)
        pltpu.sync_copy(x.at[rows, pl.ds(_HA, _HB)], idxb_v)

        def fire(r, bufA, bufB, semA, semB):
            pltpu.async_copy(table.at[idxa_v.at[r]], bufA, semA)
            pltpu.async_copy(table.at[idxb_v.at[r]], bufB, semB)

        def drain(bufA, bufB, semA, semB):
            pltpu.make_async_copy(table.at[idxa_v.at[0]], bufA, semA).wait()
            pltpu.make_async_copy(table.at[idxb_v.at[0]], bufB, semB).wait()

        def accum(r, bufA, bufB):
            z = jnp.zeros((2 * 16,), jnp.bfloat16)

            @plsc.parallel_loop(0, _HA, step=8, carry=(z,) * 4)
            def accA(j, c4):
                p = list(c4)
                for t in range(8):
                    v = plsc.bitcast(bufA[j + t, 0:16], jnp.bfloat16)
                    p[t % 4] = p[t % 4] + v
                return tuple(p)

            @plsc.parallel_loop(0, _HB, step=8, carry=accA)
            def accB(j, c4):
                p = list(c4)
                for t in range(8):
                    v = plsc.bitcast(bufB[j + t, 0:16], jnp.bfloat16)
                    p[t % 4] = p[t % 4] + v
                return tuple(p)

            p0, p1, p2, p3 = accB
            stot = (p0 + p1) + (p2 + p3)
            a, b2 = plsc.unpack(stot, format=plsc.PackFormat.INTERLEAVED)
            out_v[pl.ds(r * _EMBED, 16)] = a
            out_v[pl.ds(r * _EMBED + 16, 16)] = b2

        fire(0, bufA0, bufB0, semA0, semB0)

        def body(i, carry):
            r0 = 2 * i
            fire(r0 + 1, bufA1, bufB1, semA1, semB1)
            drain(bufA0, bufB0, semA0, semB0)
            accum(r0, bufA0, bufB0)

            @pl.when(i < _RPW // 2 - 1)
            def _():
                fire(r0 + 2, bufA0, bufB0, semA0, semB0)

            drain(bufA1, bufB1, semA1, semB1)
            accum(r0 + 1, bufA1, bufB1)
            return carry

        lax.fori_loop(0, _RPW // 2, body, 0)
        pltpu.sync_copy(out_v, out.at[pl.ds(wid * _RPW * _EMBED,
                                            _RPW * _EMBED)])

    return pool


_pool_kernel = _make_pool_kernel()


_VOCAB = 1000000
_NCOLS = _VOCAB // 128
_TAILROWS = _VOCAB - _NCOLS * 128
_UL = 256
_NUNITS = _NCOLS * 128 // _UL
_KTOT = 124


def _make_detile_kernel():
    mesh = plsc.VectorSubcoreMesh(core_axis_name="c", subcore_axis_name="s")

    @functools.partial(
        pl.kernel,
        mesh=mesh,
        compiler_params=pltpu.CompilerParams(use_tc_tiling_on_sc=True,
                                             needs_layout_passes=False),
        out_type=jax.ShapeDtypeStruct((_VOCAB * _EMBED // 2,), jnp.int32),
        scratch_types=[
            pltpu.VMEM((_EMBED, _UL), jnp.float32),
            pltpu.VMEM((_EMBED, _UL), jnp.float32),
            pltpu.VMEM((_UL * _EMBED // 2,), jnp.int32),
            pltpu.VMEM((_UL * _EMBED // 2,), jnp.int32),
            pltpu.VMEM((_UL * 33,), jnp.float32),
            pltpu.VMEM((_TAILROWS * _EMBED,), jnp.float32),
            pltpu.VMEM((_TAILROWS * _EMBED // 2,), jnp.int32),
            pltpu.SemaphoreType.DMA,
            pltpu.SemaphoreType.DMA,
            pltpu.SemaphoreType.DMA,
            pltpu.SemaphoreType.DMA,
        ],
    )
    def detile(tT, tail, out, tile0, tile1, stage0, stage1, skew_v, tailb,
               tailI, si0, si1, so0, so1):
        wid = lax.axis_index("s") * _NC + lax.axis_index("c")
        lane33 = lax.iota(jnp.int32, 16) * 33

        def colof(k):
            return jnp.minimum(k * _NW + wid, _NUNITS - 1)

        def fire_in(u, tile_v, sem):
            pltpu.async_copy(tT.at[:, pl.ds(u * _UL, _UL)], tile_v, sem)

        def wait_in(tile_v, sem):
            pltpu.make_async_copy(tT.at[:, pl.ds(0, _UL)], tile_v,
                                  sem).wait()

        def fire_out(u, stage_v, sem):
            off = pl.multiple_of(u * (_UL * _EMBED // 2), 8)
            pltpu.async_copy(stage_v,
                             out.at[pl.ds(off, _UL * _EMBED // 2)], sem)

        def wait_out(stage_v, sem):
            pltpu.make_async_copy(stage_v,
                                  out.at[pl.ds(0, _UL * _EMBED // 2)],
                                  sem).wait()

        def shuffle(tile_v, stage_v):
            @plsc.parallel_loop(0, _EMBED, unroll=4)
            def _(c):
                for g in range(_UL // 16):
                    v = tile_v[c, 16 * g:16 * (g + 1)]
                    plsc.store_scatter(skew_v, [lane33 + (33 * 16 * g) + c],
                                       v)

            @plsc.parallel_loop(0, _UL, unroll=8)
            def _(r):
                a = skew_v[pl.ds(r * 33, 16)]
                b = skew_v[pl.ds(r * 33 + 16, 16)]
                packed = plsc.pack(a, b, format=plsc.PackFormat.INTERLEAVED)
                stage_v[pl.ds(r * 16, 16)] = plsc.bitcast(packed, jnp.int32)

        fire_in(colof(0), tile0, si0)
        fire_in(colof(1), tile1, si1)

        def body(i, carry):
            k0 = 2 * i
            wait_in(tile0, si0)

            @pl.when(i > 0)
            def _():
                wait_out(stage0, so0)

            shuffle(tile0, stage0)
            fire_out(colof(k0), stage0, so0)
            fire_in(colof(k0 + 2), tile0, si0)

            wait_in(tile1, si1)

            @pl.when(i > 0)
            def _():
                wait_out(stage1, so1)

            shuffle(tile1, stage1)
            fire_out(colof(k0 + 1), stage1, so1)
            fire_in(colof(k0 + 3), tile1, si1)
            return carry

        lax.fori_loop(0, _KTOT // 2, body, 0)
        wait_in(tile0, si0)
        wait_in(tile1, si1)
        wait_out(stage0, so0)
        wait_out(stage1, so1)

        @pl.when(wid == 0)
        def _():
            pltpu.sync_copy(tail, tailb)

            @plsc.parallel_loop(0, _TAILROWS, unroll=4)
            def _(r):
                a = tailb[pl.ds(r * 32, 16)]
                bb = tailb[pl.ds(r * 32 + 16, 16)]
                packed = plsc.pack(a, bb,
                                   format=plsc.PackFormat.INTERLEAVED)
                tailI[pl.ds(r * 16, 16)] = plsc.bitcast(packed, jnp.int32)

            pltpu.sync_copy(
                tailI, out.at[pl.ds(_NCOLS * 128 * _EMBED // 2,
                                    _TAILROWS * _EMBED // 2)])

    return detile


_detile_kernel = _make_detile_kernel()


def _mm_body(p_ref, w_ref, b_ref, o_ref):
    o_ref[...] = (
        jnp.dot(p_ref[...], w_ref[...], preferred_element_type=jnp.float32)
        + b_ref[...]
    )


_ILEAVE = [v for i in range(16) for v in (i, 16 + i)]


def kernel(x, table, W, b):
    xi = x.astype(jnp.int32)
    tail = table[_NCOLS * 128:].reshape(_TAILROWS * _EMBED)
    t16 = _detile_kernel(table.T, tail).reshape(_VOCAB, _EMBED // 2)
    pooled = _pool_kernel(xi, t16).reshape(_BATCH, _EMBED)
    wt = (W.T / float(_HIST)).astype(jnp.float32)
    out = pl.pallas_call(
        _mm_body,
        out_shape=jax.ShapeDtypeStruct((_BATCH, _OUT), jnp.float32),
    )(pooled, wt, b.reshape(1, _OUT))
    return out


# speedup vs baseline: 9.0527x; 1.0226x over previous
import functools

import jax
import jax.numpy as jnp
from jax import lax
from jax.experimental import pallas as pl
from jax.experimental.pallas import tpu as pltpu
from jax.experimental.pallas import tpu_sc as plsc

_BATCH = 4096
_HIST = 200
_EMBED = 32
_OUT = 16
_NC = 2
_NS = 16
_NW = _NC * _NS
_RPW = _BATCH // _NW
_HA = 128
_HB = _HIST - _HA


def _make_pool_kernel():
    mesh = plsc.VectorSubcoreMesh(core_axis_name="c", subcore_axis_name="s")

    @functools.partial(
        pl.kernel,
        mesh=mesh,
        compiler_params=pltpu.CompilerParams(use_tc_tiling_on_sc=False,
                                             needs_layout_passes=False),
        out_type=jax.ShapeDtypeStruct((_BATCH * _EMBED,), jnp.float32),
        scratch_types=[
            pltpu.VMEM((_RPW, _HA), jnp.int32),
            pltpu.VMEM((_RPW, _HB), jnp.int32),
            pltpu.VMEM((_HA, _EMBED // 2), jnp.int32),
            pltpu.VMEM((_HA, _EMBED // 2), jnp.int32),
            pltpu.VMEM((_HB, _EMBED // 2), jnp.int32),
            pltpu.VMEM((_HB, _EMBED // 2), jnp.int32),
            pltpu.VMEM((_RPW * _EMBED,), jnp.float32),
            pltpu.SemaphoreType.DMA,
            pltpu.SemaphoreType.DMA,
            pltpu.SemaphoreType.DMA,
            pltpu.SemaphoreType.DMA,
        ],
    )
    def pool(x, table, out, idxa_v, idxb_v, bufA0, bufA1, bufB0,
             bufB1, out_v, semA0, semA1, semB0, semB1):
        wid = lax.axis_index("s") * _NC + lax.axis_index("c")

        rows = pl.ds(wid * _RPW, _RPW)
        pltpu.sync_copy(x.at[rows, pl.ds(0, _HA)], idxa_v)
        pltpu.sync_copy(x.at[rows, pl.ds(_HA, _HB)], idxb_v)

        def fire(r, bufA, bufB, semA, semB):
            pltpu.async_copy(table.at[idxa_v.at[r]], bufA, semA)
            pltpu.async_copy(table.at[idxb_v.at[r]], bufB, semB)

        def drain(bufA, bufB, semA, semB):
            pltpu.make_async_copy(table.at[idxa_v.at[0]], bufA, semA).wait()
            pltpu.make_async_copy(table.at[idxb_v.at[0]], bufB, semB).wait()

        def accum(r, bufA, bufB):
            z = jnp.zeros((2 * 16,), jnp.bfloat16)

            @plsc.parallel_loop(0, _HA, step=8, carry=(z,) * 4)
            def accA(j, c4):
                p = list(c4)
                for t in range(8):
                    v = plsc.bitcast(bufA[j + t, 0:16], jnp.bfloat16)
                    p[t % 4] = p[t % 4] + v
                return tuple(p)

            @plsc.parallel_loop(0, _HB, step=8, carry=accA)
            def accB(j, c4):
                p = list(c4)
                for t in range(8):
                    v = plsc.bitcast(bufB[j + t, 0:16], jnp.bfloat16)
                    p[t % 4] = p[t % 4] + v
                return tuple(p)

            p0, p1, p2, p3 = accB
            stot = (p0 + p1) + (p2 + p3)
            a, b2 = plsc.unpack(stot, format=plsc.PackFormat.INTERLEAVED)
            out_v[pl.ds(r * _EMBED, 16)] = a
            out_v[pl.ds(r * _EMBED + 16, 16)] = b2

        fire(0, bufA0, bufB0, semA0, semB0)

        def body(i, carry):
            r0 = 2 * i
            fire(r0 + 1, bufA1, bufB1, semA1, semB1)
            drain(bufA0, bufB0, semA0, semB0)
            accum(r0, bufA0, bufB0)

            @pl.when(i < _RPW // 2 - 1)
            def _():
                fire(r0 + 2, bufA0, bufB0, semA0, semB0)

            drain(bufA1, bufB1, semA1, semB1)
            accum(r0 + 1, bufA1, bufB1)
            return carry

        lax.fori_loop(0, _RPW // 2, body, 0)
        pltpu.sync_copy(out_v, out.at[pl.ds(wid * _RPW * _EMBED,
                                            _RPW * _EMBED)])

    return pool


_pool_kernel = _make_pool_kernel()


_VOCAB = 1000000
_NCOLS = _VOCAB // 128
_TAILROWS = _VOCAB - _NCOLS * 128
_UL = 512
_NUNITS = _NCOLS * 128 // _UL
_KTOT = 62


def _make_detile_kernel():
    mesh = plsc.VectorSubcoreMesh(core_axis_name="c", subcore_axis_name="s")

    @functools.partial(
        pl.kernel,
        mesh=mesh,
        compiler_params=pltpu.CompilerParams(use_tc_tiling_on_sc=True,
                                             needs_layout_passes=False),
        out_type=jax.ShapeDtypeStruct((_VOCAB * _EMBED // 2,), jnp.int32),
        scratch_types=[
            pltpu.VMEM((_EMBED, _UL), jnp.float32),
            pltpu.VMEM((_EMBED, _UL), jnp.float32),
            pltpu.VMEM((_UL * _EMBED // 2,), jnp.int32),
            pltpu.VMEM((_UL * _EMBED // 2,), jnp.int32),
            pltpu.VMEM((_UL * 33,), jnp.float32),
            pltpu.VMEM((_TAILROWS * _EMBED,), jnp.float32),
            pltpu.VMEM((_TAILROWS * _EMBED // 2,), jnp.int32),
            pltpu.SemaphoreType.DMA,
            pltpu.SemaphoreType.DMA,
            pltpu.SemaphoreType.DMA,
            pltpu.SemaphoreType.DMA,
        ],
    )
    def detile(tT, tail, out, tile0, tile1, stage0, stage1, skew_v, tailb,
               tailI, si0, si1, so0, so1):
        wid = lax.axis_index("s") * _NC + lax.axis_index("c")
        lane33 = lax.iota(jnp.int32, 16) * 33

        def colof(k):
            return jnp.minimum(k * _NW + wid, _NUNITS - 1)

        def fire_in(u, tile_v, sem):
            pltpu.async_copy(tT.at[:, pl.ds(u * _UL, _UL)], tile_v, sem)

        def wait_in(tile_v, sem):
            pltpu.make_async_copy(tT.at[:, pl.ds(0, _UL)], tile_v,
                                  sem).wait()

        def fire_out(u, stage_v, sem):
            off = pl.multiple_of(u * (_UL * _EMBED // 2), 8)
            pltpu.async_copy(stage_v,
                             out.at[pl.ds(off, _UL * _EMBED // 2)], sem)

        def wait_out(stage_v, sem):
            pltpu.make_async_copy(stage_v,
                                  out.at[pl.ds(0, _UL * _EMBED // 2)],
                                  sem).wait()

        def shuffle(tile_v, stage_v):
            @plsc.parallel_loop(0, _EMBED, unroll=4)
            def _(c):
                for g in range(_UL // 16):
                    v = tile_v[c, 16 * g:16 * (g + 1)]
                    plsc.store_scatter(skew_v, [lane33 + (33 * 16 * g) + c],
                                       v)

            @plsc.parallel_loop(0, _UL, unroll=8)
            def _(r):
                a = skew_v[pl.ds(r * 33, 16)]
                b = skew_v[pl.ds(r * 33 + 16, 16)]
                packed = plsc.pack(a, b, format=plsc.PackFormat.INTERLEAVED)
                stage_v[pl.ds(r * 16, 16)] = plsc.bitcast(packed, jnp.int32)

        fire_in(colof(0), tile0, si0)
        fire_in(colof(1), tile1, si1)

        def body(i, carry):
            k0 = 2 * i
            wait_in(tile0, si0)

            @pl.when(i > 0)
            def _():
                wait_out(stage0, so0)

            shuffle(tile0, stage0)
            fire_out(colof(k0), stage0, so0)
            fire_in(colof(k0 + 2), tile0, si0)

            wait_in(tile1, si1)

            @pl.when(i > 0)
            def _():
                wait_out(stage1, so1)

            shuffle(tile1, stage1)
            fire_out(colof(k0 + 1), stage1, so1)
            fire_in(colof(k0 + 3), tile1, si1)
            return carry

        lax.fori_loop(0, _KTOT // 2, body, 0)
        wait_in(tile0, si0)
        wait_in(tile1, si1)
        wait_out(stage0, so0)
        wait_out(stage1, so1)

        @pl.when(wid == 0)
        def _():
            pltpu.sync_copy(tail, tailb)

            @plsc.parallel_loop(0, _TAILROWS, unroll=4)
            def _(r):
                a = tailb[pl.ds(r * 32, 16)]
                bb = tailb[pl.ds(r * 32 + 16, 16)]
                packed = plsc.pack(a, bb,
                                   format=plsc.PackFormat.INTERLEAVED)
                tailI[pl.ds(r * 16, 16)] = plsc.bitcast(packed, jnp.int32)

            pltpu.sync_copy(
                tailI, out.at[pl.ds(_NCOLS * 128 * _EMBED // 2,
                                    _TAILROWS * _EMBED // 2)])

    return detile


_detile_kernel = _make_detile_kernel()


def _mm_body(p_ref, w_ref, b_ref, o_ref):
    o_ref[...] = (
        jnp.dot(p_ref[...], w_ref[...], preferred_element_type=jnp.float32)
        + b_ref[...]
    )


_ILEAVE = [v for i in range(16) for v in (i, 16 + i)]


def kernel(x, table, W, b):
    xi = x.astype(jnp.int32)
    tail = table[_NCOLS * 128:].reshape(_TAILROWS * _EMBED)
    t16 = _detile_kernel(table.T, tail).reshape(_VOCAB, _EMBED // 2)
    pooled = _pool_kernel(xi, t16).reshape(_BATCH, _EMBED)
    wt = (W.T / float(_HIST)).astype(jnp.float32)
    out = pl.pallas_call(
        _mm_body,
        out_shape=jax.ShapeDtypeStruct((_BATCH, _OUT), jnp.float32),
    )(pooled, wt, b.reshape(1, _OUT))
    return out
